# baseline (speedup 1.0000x reference)
import functools

import jax
import jax.numpy as jnp
from jax import lax
from jax.experimental import pallas as pl
from jax.experimental.pallas import tpu as pltpu
from jax.experimental.pallas import tpu_sc as plsc

D_MODEL = 1024
B_TOK = 4 * 8192
NC, NS = 2, 16
NW = NC * NS
B_PER_W = B_TOK // NW
L = 16
N_GROUPS = B_PER_W // L
ND = D_MODEL // L


@functools.partial(
    pl.kernel,
    out_type=jax.ShapeDtypeStruct((B_TOK, D_MODEL), jnp.float32),
    mesh=plsc.VectorSubcoreMesh(core_axis_name="c", subcore_axis_name="s"),
    scratch_types=[
        pltpu.VMEM((B_PER_W,), jnp.int32),
        pltpu.VMEM((2, D_MODEL), jnp.float32),
        pltpu.VMEM((2 * L, 1, D_MODEL), jnp.float32),
        pltpu.SemaphoreType.DMA,
    ],
)
def _embed_lookup(sid_hbm, table_hbm, out_hbm, idx_v, table_v, rep, sem):
    wid = lax.axis_index("s") * NC + lax.axis_index("c")
    base = wid * B_PER_W
    row = wid // 8
    col = (wid % 8) * B_PER_W

    idx_cp = pltpu.async_copy(sid_hbm.at[row, pl.ds(col, B_PER_W)], idx_v, sem)
    pltpu.sync_copy(table_hbm, table_v)

    def fill_body(r, carry):
        for d in range(ND):
            sl = pl.ds(d * L, L)
            rep[r, 0, sl] = table_v[0, sl]
            rep[L + r, 0, sl] = table_v[1, sl]
        return carry

    lax.fori_loop(0, L, fill_body, 0)

    idx_cp.wait()
    iot = lax.iota(jnp.int32, L)

    def group_body(g, carry):
        off = pl.multiple_of(g * L, L)
        sv = idx_v[pl.ds(off, L)]
        pos = iot + (base + off)
        comp = sv * 65536 + pos
        for lgk in (1, 2, 3, 4):
            for lgj in range(lgk - 1, -1, -1):
                cp = comp[iot ^ (1 << lgj)]
                takemin = (((iot >> lgj) & 1) ^ ((iot >> lgk) & 1)) == 0
                comp = jnp.where(takemin, jnp.minimum(comp, cp),
                                 jnp.maximum(comp, cp))
        dvec = comp & 65535
        ps = sv
        for sft in (1, 2, 4, 8):
            shifted = ps[jnp.maximum(iot - sft, 0)]
            ps = ps + jnp.where(iot >= sft, shifted, 0)
        n1 = ps[L - 1]
        pltpu.async_copy(rep.at[pl.ds(n1, L), 0], out_hbm.at[dvec], sem)
        return carry

    lax.fori_loop(0, N_GROUPS, group_body, 0)

    def drain_body(g, carry):
        pltpu.make_async_copy(
            out_hbm.at[pl.ds(0, L)], rep.at[pl.ds(0, L), 0], sem).wait()
        return carry

    lax.fori_loop(0, N_GROUPS, drain_body, 0)


def kernel(segment_ids, emb_weight):
    sid = segment_ids.astype(jnp.int32)
    out = _embed_lookup(sid, emb_weight)
    return out.reshape(segment_ids.shape[0], segment_ids.shape[1], D_MODEL)


# speedup vs baseline: 1.0502x; 1.0502x over previous
import functools

import jax
import jax.numpy as jnp
from jax import lax
from jax.experimental import pallas as pl
from jax.experimental.pallas import tpu as pltpu
from jax.experimental.pallas import tpu_sc as plsc

D_MODEL = 1024
B_TOK = 4 * 8192
NC, NS = 2, 16
NW = NC * NS
B_PER_W = B_TOK // NW
L = 16
N_GROUPS = B_PER_W // L
ND = D_MODEL // L


@functools.partial(
    pl.kernel,
    out_type=jax.ShapeDtypeStruct((B_TOK, D_MODEL), jnp.float32),
    mesh=plsc.VectorSubcoreMesh(core_axis_name="c", subcore_axis_name="s"),
    scratch_types=[
        pltpu.VMEM((B_PER_W,), jnp.int32),
        pltpu.VMEM((2 * L, 1, D_MODEL), jnp.float32),
        pltpu.SemaphoreType.DMA,
    ],
)
def _embed_lookup(sid_hbm, rep_hbm, out_hbm, idx_v, rep, sem):
    wid = lax.axis_index("s") * NC + lax.axis_index("c")
    base = wid * B_PER_W
    row = wid // 8
    col = (wid % 8) * B_PER_W

    idx_cp = pltpu.async_copy(sid_hbm.at[row, pl.ds(col, B_PER_W)], idx_v, sem)
    pltpu.sync_copy(rep_hbm, rep)

    idx_cp.wait()
    iot = lax.iota(jnp.int32, L)

    def group_body(g, carry):
        off = pl.multiple_of(g * L, L)
        sv = idx_v[pl.ds(off, L)]
        pos = iot + (base + off)
        comp = sv * 65536 + pos
        for lgk in (1, 2, 3, 4):
            for lgj in range(lgk - 1, -1, -1):
                cp = comp[iot ^ (1 << lgj)]
                takemin = (((iot >> lgj) & 1) ^ ((iot >> lgk) & 1)) == 0
                comp = jnp.where(takemin, jnp.minimum(comp, cp),
                                 jnp.maximum(comp, cp))
        dvec = comp & 65535
        ps = sv
        for sft in (1, 2, 4, 8):
            shifted = ps[jnp.maximum(iot - sft, 0)]
            ps = ps + jnp.where(iot >= sft, shifted, 0)
        n1 = ps[L - 1]
        pltpu.async_copy(rep.at[pl.ds(n1, L), 0], out_hbm.at[dvec], sem)
        return carry

    lax.fori_loop(0, N_GROUPS, group_body, 0)

    pltpu.make_async_copy(
        out_hbm.at[pl.ds(base, B_PER_W)],
        out_hbm.at[pl.ds(base, B_PER_W)], sem).wait()


def kernel(segment_ids, emb_weight):
    sid = segment_ids.astype(jnp.int32)
    rep_hbm = jnp.repeat(emb_weight, L, axis=0).reshape(2 * L, 1, D_MODEL)
    out = _embed_lookup(sid, rep_hbm)
    return out.reshape(segment_ids.shape[0], segment_ids.shape[1], D_MODEL)
